# baseline (speedup 1.0000x reference)
import functools

import jax
import jax.numpy as jnp
from jax import lax
from jax.experimental import pallas as pl
from jax.experimental.pallas import tpu as pltpu
from jax.experimental.pallas import tpu_sc as plsc

_B, _N, _C, _K, _TOPK = 2, 2048, 1024, 8, 2
_EPS = 1e-06
_RATIO = 0.1
_LB_W = 0.01

_GATE_BN = 512
_BNE = 512
_P = _B * _N * _TOPK
_NW = 32
_CH = 4
_RPC = _P // (_NW * _CH)


def _gate_kernel(tok_ref, geno_ref, Wg_ref, bg_ref, Wgg_ref, bgg_ref,
                 lb_ref, cnt_ref, dst_ref, src_ref, val_ref,
                 cnts_ref, off_ref):
    b = pl.program_id(0)
    nb = pl.program_id(1)
    nblocks = pl.num_programs(1)
    bn = _GATE_BN

    tok = tok_ref[0]
    lg = jnp.dot(tok, Wg_ref[...], preferred_element_type=jnp.float32)
    g = jnp.dot(geno_ref[0], Wgg_ref[...], preferred_element_type=jnp.float32)
    lg = lg + bg_ref[...] + _RATIO * (g + bgg_ref[...])

    iota = lax.broadcasted_iota(jnp.int32, lg.shape, 1)
    m1 = jnp.max(lg, axis=1, keepdims=True)
    i1 = jnp.min(jnp.where(lg == m1, iota, _K), axis=1, keepdims=True)
    oh1 = iota == i1
    lg2 = jnp.where(oh1, jnp.float32(-1e30), lg)
    m2 = jnp.max(lg2, axis=1, keepdims=True)
    i2 = jnp.min(jnp.where(lg2 == m2, iota, _K), axis=1, keepdims=True)
    oh2 = iota == i2

    e2 = jnp.exp(m2 - m1)
    denom = 1.0 + e2
    w1 = jnp.maximum(1.0 / denom, _EPS)
    w2 = jnp.maximum(e2 / denom, _EPS)
    s = w1 + w2
    w1 = w1 / s
    w2 = w2 / s

    oh1f = oh1.astype(jnp.float32)
    oh2f = oh2.astype(jnp.float32)
    hit = oh1f + oh2f

    r_iota = lax.broadcasted_iota(jnp.int32, (bn, bn), 0)
    c_iota = lax.broadcasted_iota(jnp.int32, (bn, bn), 1)
    lstrict = (r_iota > c_iota).astype(jnp.float32)
    cnt_before = jnp.dot(lstrict, hit, preferred_element_type=jnp.float32)

    @pl.when(nb == 0)
    def _():
        off_ref[...] = jnp.zeros_like(off_ref)

    rank = cnt_before + off_ref[...]
    base = ((b * _K + iota) * _N).astype(jnp.float32)
    poslane = base + rank
    pos1 = jnp.sum(oh1f * poslane, axis=1, keepdims=True)
    pos2 = jnp.sum(oh2f * poslane, axis=1, keepdims=True)
    dstT = jnp.concatenate([pos1, pos2], axis=1).T
    dst_ref[0] = dstT.astype(jnp.int32)

    srow = (b * _N + nb * bn
            + lax.broadcasted_iota(jnp.int32, (2, bn), 1))
    src_ref[0] = srow

    valT = jnp.concatenate([w1, w2], axis=1).T
    val_ref[0] = jnp.broadcast_to(valT[:, :, None], (2, bn, 128))

    off_new = off_ref[...] + jnp.sum(hit, axis=0, keepdims=True)
    off_ref[...] = off_new
    cnts_ref[0] = off_new

    @pl.when((b == 0) & (nb == 0))
    def _():
        cnt_ref[...] = jnp.zeros_like(cnt_ref)

    cnt_ref[...] += jnp.sum(hit, axis=0, keepdims=True)

    @pl.when((b == pl.num_programs(0) - 1) & (nb == nblocks - 1))
    def _():
        usage = cnt_ref[...] * (1.0 / (_B * _N))
        m = jnp.mean(usage)
        v = jnp.mean((usage - m) ** 2)
        lb_ref[...] = (_LB_W * v / (m + _EPS) ** 2).reshape(1, 1)


def _sc_dispatch(tok_hbm, src_hbm, dst_hbm, val_hbm, G_hbm, VS_hbm,
                 gidx_v, sidx_v, rows_v, vs_v, sem):
    wid = lax.axis_index("s") * 2 + lax.axis_index("c")
    for c in range(_CH):
        row = wid * _CH + c
        pltpu.sync_copy(src_hbm.at[row], gidx_v)
        pltpu.async_copy(tok_hbm.at[gidx_v], rows_v, sem).wait()
        pltpu.sync_copy(dst_hbm.at[row], sidx_v)
        pltpu.async_copy(rows_v, G_hbm.at[sidx_v], sem).wait()
        pltpu.sync_copy(val_hbm.at[row], vs_v)
        pltpu.async_copy(vs_v, VS_hbm.at[sidx_v], sem).wait()


def _expert_kernel(cnt_sref, vs_ref, g_ref, W1_ref, b1_ref, W2_ref, b2_ref,
                   out_ref, hacc_ref, wacc_ref, w1bf_ref):
    k = pl.program_id(0)
    b = pl.program_id(1)
    j = pl.program_id(2)
    jmax = pl.num_programs(2)
    cnt = cnt_sref[b * _K + k]

    @pl.when((b == 0) & (j == 0))
    def _():
        w1bf_ref[...] = W1_ref[0].astype(jnp.bfloat16)

    @pl.when(j == 0)
    def _():
        hacc_ref[...] = jnp.zeros_like(hacc_ref)
        wacc_ref[0, 0] = 0.0

    @pl.when(j * _BNE < cnt)
    def _():
        rows = g_ref[0].astype(jnp.bfloat16)
        h = jnp.dot(rows, w1bf_ref[...],
                    preferred_element_type=jnp.float32)
        h = jnp.maximum(h + b1_ref[0], 0.0)
        wv = vs_ref[0][:, 0:1]
        riota = lax.broadcasted_iota(jnp.int32, (_BNE, 1), 0)
        mask = (j * _BNE + riota) < cnt
        hw = jnp.where(mask, h * wv, 0.0)
        hacc_ref[...] += jnp.sum(hw.reshape(_BNE // 8, 8, _C), axis=0)
        wacc_ref[0, 0] += jnp.sum(jnp.where(mask, wv, 0.0))

    @pl.when(j == jmax - 1)
    def _():
        ws = wacc_ref[0, 0]
        hsum = jnp.sum(hacc_ref[...], axis=0, keepdims=True)
        num = jnp.dot(hsum, W2_ref[0],
                      preferred_element_type=jnp.float32) + ws * b2_ref[0]
        out_ref[0] = num / (ws + _EPS)


def _active_block(j, cnt):
    jact = (cnt + _BNE - 1) // _BNE
    return jnp.minimum(j, jnp.maximum(jact - 1, 0))


@jax.jit
def kernel(tokens, geno_vec, Wg, bg, Wgg, bgg, W1, b1, W2, b2):
    B, N, C, K = _B, _N, _C, _K
    gnb = N // _GATE_BN

    lb, _cnt, dst, src, val, cnts = pl.pallas_call(
        _gate_kernel,
        grid=(B, gnb),
        in_specs=[
            pl.BlockSpec((1, _GATE_BN, C), lambda b, n: (b, n, 0)),
            pl.BlockSpec((1, 1, C), lambda b, n: (b, 0, 0)),
            pl.BlockSpec((C, K), lambda b, n: (0, 0)),
            pl.BlockSpec((1, K), lambda b, n: (0, 0)),
            pl.BlockSpec((C, K), lambda b, n: (0, 0)),
            pl.BlockSpec((1, K), lambda b, n: (0, 0)),
        ],
        out_specs=[
            pl.BlockSpec((1, 1), lambda b, n: (0, 0)),
            pl.BlockSpec((1, K), lambda b, n: (0, 0)),
            pl.BlockSpec((1, 2, _GATE_BN), lambda b, n: (b, 0, n)),
            pl.BlockSpec((1, 2, _GATE_BN), lambda b, n: (b, 0, n)),
            pl.BlockSpec((1, 2, _GATE_BN, 128), lambda b, n: (b, 0, n, 0)),
            pl.BlockSpec((1, 1, K), lambda b, n: (b, 0, 0)),
        ],
        out_shape=[
            jax.ShapeDtypeStruct((1, 1), jnp.float32),
            jax.ShapeDtypeStruct((1, K), jnp.float32),
            jax.ShapeDtypeStruct((B, 2, N), jnp.int32),
            jax.ShapeDtypeStruct((B, 2, N), jnp.int32),
            jax.ShapeDtypeStruct((B, 2, N, 128), jnp.float32),
            jax.ShapeDtypeStruct((B, 1, K), jnp.float32),
        ],
        scratch_shapes=[pltpu.VMEM((1, K), jnp.float32)],
    )(tokens, geno_vec.reshape(B, 1, C), Wg, bg.reshape(1, K),
      Wgg, bgg.reshape(1, K))

    tok3 = tokens.reshape(B * N, 8, C // 8)
    srcr = src.reshape(_NW * _CH, _RPC)
    dstr = dst.reshape(_NW * _CH, _RPC)
    valr = val.reshape(_NW * _CH, _RPC, 128)

    mesh = plsc.VectorSubcoreMesh(core_axis_name="c", subcore_axis_name="s")
    G, VS = pl.kernel(
        _sc_dispatch,
        out_type=[
            jax.ShapeDtypeStruct((B * K * N, 8, C // 8), jnp.float32),
            jax.ShapeDtypeStruct((B * K * N, 128), jnp.float32),
        ],
        mesh=mesh,
        scratch_types=[
            pltpu.VMEM((_RPC,), jnp.int32),
            pltpu.VMEM((_RPC,), jnp.int32),
            pltpu.VMEM((_RPC, 8, C // 8), jnp.float32),
            pltpu.VMEM((_RPC, 128), jnp.float32),
            pltpu.SemaphoreType.DMA,
        ],
    )(tok3, srcr, dstr, valr)

    cnt_i = cnts.reshape(B * K).astype(jnp.int32)
    g3 = G.reshape(B * K, N, C)
    vs3 = VS.reshape(B * K, N, 128)
    jblk = N // _BNE

    centers = pl.pallas_call(
        _expert_kernel,
        grid_spec=pltpu.PrefetchScalarGridSpec(
            num_scalar_prefetch=1,
            grid=(K, B, jblk),
            in_specs=[
                pl.BlockSpec(
                    (1, _BNE, 128),
                    lambda k, b, j, s: (b * _K + k,
                                        _active_block(j, s[b * _K + k]), 0)),
                pl.BlockSpec(
                    (1, _BNE, C),
                    lambda k, b, j, s: (b * _K + k,
                                        _active_block(j, s[b * _K + k]), 0)),
                pl.BlockSpec((1, C, C), lambda k, b, j, s: (k, 0, 0)),
                pl.BlockSpec((1, 1, C), lambda k, b, j, s: (k, 0, 0)),
                pl.BlockSpec((1, C, C), lambda k, b, j, s: (k, 0, 0)),
                pl.BlockSpec((1, 1, C), lambda k, b, j, s: (k, 0, 0)),
            ],
            out_specs=pl.BlockSpec(
                (1, 1, C), lambda k, b, j, s: (b * _K + k, 0, 0)),
            scratch_shapes=[
                pltpu.VMEM((8, C), jnp.float32),
                pltpu.SMEM((1, 1), jnp.float32),
                pltpu.VMEM((C, C), jnp.bfloat16),
            ],
        ),
        out_shape=jax.ShapeDtypeStruct((B * K, 1, C), jnp.float32),
    )(cnt_i, vs3, g3, W1, b1.reshape(K, 1, C), W2, b2.reshape(K, 1, C))

    return centers.reshape(B, K, C), lb.reshape(())


# speedup vs baseline: 2.7692x; 2.7692x over previous
import jax
import jax.numpy as jnp
from jax import lax
from jax.experimental import pallas as pl
from jax.experimental.pallas import tpu as pltpu

_B, _N, _C, _K, _TOPK = 2, 2048, 1024, 8, 2
_EPS = 1e-06
_RATIO = 0.1
_LB_W = 0.01


def _moe_kernel(tok_ref, geno_ref, Wg_ref, bg_ref, Wgg_ref, bgg_ref,
                W1_ref, b1_ref, W2_ref, b2_ref,
                out_ref, lb_ref, tbf_ref, wt_ref, cnt_ref, w1bf_ref):
    p = pl.program_id(0)
    b = pl.program_id(1)

    @pl.when(p == 0)
    def _gate():
        tok = tok_ref[0]
        tbf_ref[b] = tok.astype(jnp.bfloat16)
        lg = jnp.dot(tok, Wg_ref[...], preferred_element_type=jnp.float32)
        g = jnp.dot(geno_ref[0], Wgg_ref[...],
                    preferred_element_type=jnp.float32)
        lg = lg + bg_ref[...] + _RATIO * (g + bgg_ref[...])

        iota = lax.broadcasted_iota(jnp.int32, lg.shape, 1)
        m1 = jnp.max(lg, axis=1, keepdims=True)
        i1 = jnp.min(jnp.where(lg == m1, iota, _K), axis=1, keepdims=True)
        oh1 = iota == i1
        lg2 = jnp.where(oh1, jnp.float32(-1e30), lg)
        m2 = jnp.max(lg2, axis=1, keepdims=True)
        i2 = jnp.min(jnp.where(lg2 == m2, iota, _K), axis=1, keepdims=True)
        oh2 = iota == i2

        e2 = jnp.exp(m2 - m1)
        denom = 1.0 + e2
        w1 = jnp.maximum(1.0 / denom, _EPS)
        w2 = jnp.maximum(e2 / denom, _EPS)
        s = w1 + w2
        w1 = w1 / s
        w2 = w2 / s
        w = jnp.where(oh1, w1, 0.0) + jnp.where(oh2, w2, 0.0)
        wt_ref[b] = w.T

        @pl.when(b == 0)
        def _():
            cnt_ref[...] = jnp.zeros_like(cnt_ref)

        hit = oh1.astype(jnp.float32) + oh2.astype(jnp.float32)
        cnt_ref[...] += jnp.sum(hit, axis=0, keepdims=True)

        @pl.when(b == pl.num_programs(1) - 1)
        def _():
            usage = cnt_ref[...] * (1.0 / (_B * _N))
            m = jnp.mean(usage)
            v = jnp.mean((usage - m) ** 2)
            lb_ref[...] = (_LB_W * v / (m + _EPS) ** 2).reshape(1, 1)

    @pl.when((p > 0) & (b == 0))
    def _cast():
        w1bf_ref[...] = W1_ref[0].astype(jnp.bfloat16)

    @pl.when(p > 0)
    def _expert():
        k = p - 1
        rows = tbf_ref[b]
        h = jnp.dot(rows, w1bf_ref[...],
                    preferred_element_type=jnp.float32)
        h = jnp.maximum(h + b1_ref[0], 0.0)
        wv = wt_ref[b, k].reshape(_N, 1)
        hw = h * wv
        hacc = jnp.sum(hw.reshape(_N // 8, 8, _C), axis=0)
        hsum = jnp.sum(hacc, axis=0, keepdims=True)
        ws = jnp.sum(wv)
        num = jnp.dot(hsum, W2_ref[0],
                      preferred_element_type=jnp.float32) + ws * b2_ref[0]
        out_ref[0] = num / (ws + _EPS)


@jax.jit
def kernel(tokens, geno_vec, Wg, bg, Wgg, bgg, W1, b1, W2, b2):
    B, N, C, K = _B, _N, _C, _K

    centers, lb = pl.pallas_call(
        _moe_kernel,
        grid=(K + 1, B),
        in_specs=[
            pl.BlockSpec((1, N, C), lambda p, b: (jnp.where(p == 0, b, 0), 0, 0)),
            pl.BlockSpec((1, 1, C), lambda p, b: (b, 0, 0)),
            pl.BlockSpec((C, K), lambda p, b: (0, 0)),
            pl.BlockSpec((1, K), lambda p, b: (0, 0)),
            pl.BlockSpec((C, K), lambda p, b: (0, 0)),
            pl.BlockSpec((1, K), lambda p, b: (0, 0)),
            pl.BlockSpec((1, C, C), lambda p, b: (jnp.maximum(p - 1, 0), 0, 0)),
            pl.BlockSpec((1, 1, C), lambda p, b: (jnp.maximum(p - 1, 0), 0, 0)),
            pl.BlockSpec((1, C, C), lambda p, b: (jnp.maximum(p - 1, 0), 0, 0)),
            pl.BlockSpec((1, 1, C), lambda p, b: (jnp.maximum(p - 1, 0), 0, 0)),
        ],
        out_specs=[
            pl.BlockSpec(
                (1, 1, C),
                lambda p, b: ((p > 0) * (b * _K + p - 1), 0, 0)),
            pl.BlockSpec((1, 1), lambda p, b: (0, 0)),
        ],
        out_shape=[
            jax.ShapeDtypeStruct((B * K, 1, C), jnp.float32),
            jax.ShapeDtypeStruct((1, 1), jnp.float32),
        ],
        scratch_shapes=[
            pltpu.VMEM((B, N, C), jnp.bfloat16),
            pltpu.VMEM((B, K, N), jnp.float32),
            pltpu.VMEM((1, K), jnp.float32),
            pltpu.VMEM((C, C), jnp.bfloat16),
        ],
    )(tokens, geno_vec.reshape(B, 1, C), Wg, bg.reshape(1, K),
      Wgg, bgg.reshape(1, K), W1, b1.reshape(K, 1, C), W2,
      b2.reshape(K, 1, C))

    return centers.reshape(B, K, C), lb.reshape(())


# speedup vs baseline: 2.7843x; 1.0055x over previous
import jax
import jax.numpy as jnp
from jax import lax
from jax.experimental import pallas as pl
from jax.experimental.pallas import tpu as pltpu

_B, _N, _C, _K, _TOPK = 2, 2048, 1024, 8, 2
_EPS = 1e-06
_RATIO = 0.1
_LB_W = 0.01


def _moe_kernel(tok_ref, geno_ref, Wg_ref, bg_ref, Wgg_ref, bgg_ref,
                W1_ref, b1_ref, W2_ref, b2_ref,
                out_ref, lb_ref, tbf_ref, wt_ref, cnt_ref, w1bf_ref,
                hs_ref, ws_ref):
    p = pl.program_id(0)
    b = pl.program_id(1)

    @pl.when(p == 0)
    def _gate():
        tok = tok_ref[0]
        tbf_ref[b] = tok.astype(jnp.bfloat16)
        lg = jnp.dot(tok, Wg_ref[...], preferred_element_type=jnp.float32)
        g = jnp.dot(geno_ref[0], Wgg_ref[...],
                    preferred_element_type=jnp.float32)
        lg = lg + bg_ref[...] + _RATIO * (g + bgg_ref[...])

        iota = lax.broadcasted_iota(jnp.int32, lg.shape, 1)
        m1 = jnp.max(lg, axis=1, keepdims=True)
        i1 = jnp.min(jnp.where(lg == m1, iota, _K), axis=1, keepdims=True)
        oh1 = iota == i1
        lg2 = jnp.where(oh1, jnp.float32(-1e30), lg)
        m2 = jnp.max(lg2, axis=1, keepdims=True)
        i2 = jnp.min(jnp.where(lg2 == m2, iota, _K), axis=1, keepdims=True)
        oh2 = iota == i2

        e2 = jnp.exp(m2 - m1)
        denom = 1.0 + e2
        w1 = jnp.maximum(1.0 / denom, _EPS)
        w2 = jnp.maximum(e2 / denom, _EPS)
        s = w1 + w2
        w1 = w1 / s
        w2 = w2 / s
        w = jnp.where(oh1, w1, 0.0) + jnp.where(oh2, w2, 0.0)
        wt_ref[b] = w.T

        @pl.when(b == 0)
        def _():
            cnt_ref[...] = jnp.zeros_like(cnt_ref)

        hit = oh1.astype(jnp.float32) + oh2.astype(jnp.float32)
        cnt_ref[...] += jnp.sum(hit, axis=0, keepdims=True)

        @pl.when(b == pl.num_programs(1) - 1)
        def _():
            usage = cnt_ref[...] * (1.0 / (_B * _N))
            m = jnp.mean(usage)
            v = jnp.mean((usage - m) ** 2)
            lb_ref[...] = (_LB_W * v / (m + _EPS) ** 2).reshape(1, 1)

    @pl.when((p > 0) & (b == 0))
    def _cast():
        w1bf_ref[...] = W1_ref[0].astype(jnp.bfloat16)

    @pl.when(p > 0)
    def _expert():
        k = p - 1
        rows = tbf_ref[b]
        h = jnp.dot(rows, w1bf_ref[...],
                    preferred_element_type=jnp.float32)
        h = jnp.maximum(h + b1_ref[0], 0.0)
        wv = wt_ref[b, k].reshape(_N, 1)
        hw = h * wv
        hacc = jnp.sum(hw.reshape(_N // 8, 8, _C), axis=0)
        hs_ref[b] = jnp.sum(hacc, axis=0)
        ws_ref[b] = jnp.full((128,), jnp.sum(wv), jnp.float32)

        @pl.when(b == pl.num_programs(1) - 1)
        def _():
            wsv = ws_ref[:, 0:1]
            num = jnp.dot(hs_ref[...], W2_ref[0],
                          preferred_element_type=jnp.float32)
            num = num + wsv * b2_ref[0]
            out_ref[0] = num / (wsv + _EPS)


@jax.jit
def kernel(tokens, geno_vec, Wg, bg, Wgg, bgg, W1, b1, W2, b2):
    B, N, C, K = _B, _N, _C, _K

    centers, lb = pl.pallas_call(
        _moe_kernel,
        grid=(K + 1, B),
        in_specs=[
            pl.BlockSpec((1, N, C), lambda p, b: (jnp.where(p == 0, b, 1), 0, 0)),
            pl.BlockSpec((1, 1, C), lambda p, b: (b, 0, 0)),
            pl.BlockSpec((C, K), lambda p, b: (0, 0)),
            pl.BlockSpec((1, K), lambda p, b: (0, 0)),
            pl.BlockSpec((C, K), lambda p, b: (0, 0)),
            pl.BlockSpec((1, K), lambda p, b: (0, 0)),
            pl.BlockSpec((1, C, C), lambda p, b: (jnp.maximum(p - 1, 0), 0, 0)),
            pl.BlockSpec((1, 1, C), lambda p, b: (jnp.maximum(p - 1, 0), 0, 0)),
            pl.BlockSpec((1, C, C), lambda p, b: (jnp.maximum(p - 1, 0), 0, 0)),
            pl.BlockSpec((1, 1, C), lambda p, b: (jnp.maximum(p - 1, 0), 0, 0)),
        ],
        out_specs=[
            pl.BlockSpec(
                (1, B, C),
                lambda p, b: ((p > 0) * (p - 1), 0, 0)),
            pl.BlockSpec((1, 1), lambda p, b: (0, 0)),
        ],
        out_shape=[
            jax.ShapeDtypeStruct((K, B, C), jnp.float32),
            jax.ShapeDtypeStruct((1, 1), jnp.float32),
        ],
        scratch_shapes=[
            pltpu.VMEM((B, N, C), jnp.bfloat16),
            pltpu.VMEM((B, K, N), jnp.float32),
            pltpu.VMEM((1, K), jnp.float32),
            pltpu.VMEM((C, C), jnp.bfloat16),
            pltpu.VMEM((B, C), jnp.float32),
            pltpu.VMEM((B, 128), jnp.float32),
        ],
    )(tokens, geno_vec.reshape(B, 1, C), Wg, bg.reshape(1, K),
      Wgg, bgg.reshape(1, K), W1, b1.reshape(K, 1, C), W2,
      b2.reshape(K, 1, C))

    return centers.transpose(1, 0, 2), lb.reshape(())
